# Initial kernel scaffold; baseline (speedup 1.0000x reference)
import jax
import jax.numpy as jnp
from jax.experimental import pallas as pl


def kernel(x, edge_index, edge_vals, W_a, W_b, W_c, att_w1, att_b1, att_w2):
    raise NotImplementedError("write your pallas kernel here")


# speedup vs baseline: 2.6739x; 2.6739x over previous
import functools

import jax
import jax.numpy as jnp
from jax import lax
from jax.experimental import pallas as pl
from jax.experimental.pallas import tpu as pltpu
from jax.experimental.pallas import tpu_sc as plsc

N = 10000
E = 320000
D = 128
ATT_H = 32
ALPHA = 0.5
BETA = 0.5

NC = 2
NS = 16
LANES = 16

CH = 80
NP = 10240
STRIPE = NP // NS
ZROWS = 128

def _zero_fill(zbuf):
    def body(r, _):
        for j in range(D // LANES):
            zbuf[r, pl.ds(j * LANES, LANES)] = jnp.zeros((LANES,), jnp.float32)
        return 0
    lax.fori_loop(0, ZROWS, body, 0)


def _spmm_pass(pre, ei, ev, out, acc, zbuf, src_v, dst_v, val_v, rows_v, sem,
               s, set_id, edge_base, nchunks, goff, goff_static, slot):
    for z in range(STRIPE // ZROWS):
        pltpu.sync_copy(zbuf, acc.at[pl.ds(s * STRIPE + z * ZROWS, ZROWS)])
    plsc.subcore_barrier()

    dst_off = 2 * set_id * E
    src_off = (2 * set_id + 1) * E
    val_off = set_id * E

    def chunk(cidx, _):
        base = edge_base + cidx * CH
        pltpu.sync_copy(ei.at[pl.ds(src_off + base, CH)], src_v)
        pltpu.sync_copy(ei.at[pl.ds(dst_off + base, CH)], dst_v)
        pltpu.sync_copy(ev.at[pl.ds(val_off + base, CH)], val_v)
        if goff_static is None or goff_static != 0:
            add = goff if goff_static is None else goff_static
            for k in range(CH // LANES):
                sl = pl.ds(k * LANES, LANES)
                src_v[sl] = src_v[sl] + add
        pltpu.async_copy(pre.at[src_v], rows_v, sem).wait()

        def scale(g, _):
            vv = val_v[pl.ds(g * LANES, LANES)]
            for j in range(LANES):
                v = vv[j]
                e = g * LANES + j
                for k in range(D // LANES):
                    sl = pl.ds(k * LANES, LANES)
                    rows_v[e, sl] = rows_v[e, sl] * v
            return 0
        lax.fori_loop(0, CH // LANES, scale, 0)

        pltpu.sync_copy(rows_v, acc.at[dst_v], add=True)
        return 0

    lax.fori_loop(0, nchunks, chunk, 0)
    plsc.subcore_barrier()

    pltpu.sync_copy(acc.at[pl.ds(s * STRIPE, STRIPE)],
                    out.at[pl.ds(slot * NP + s * STRIPE, STRIPE)])


def _spmm_main(pre, ei, ev, out, acc, zbuf, src_v, dst_v, val_v, rows_v, sem):
    c = lax.axis_index("c")
    s = lax.axis_index("s")
    _zero_fill(zbuf)
    for i in range(5):
        if i == 0:
            _spmm_pass(pre, ei, ev, out, acc, zbuf, src_v, dst_v, val_v,
                       rows_v, sem, s,
                       set_id=0,
                       edge_base=c * (E // 2) + s * (E // 2 // NS),
                       nchunks=E // 2 // NS // CH,
                       goff=0, goff_static=0,
                       slot=c * 9)
        else:
            pid = c * 4 + (i - 1)
            set_id = 1 + 2 * c + (i - 1) // 2
            half = (i - 1) % 2
            _spmm_pass(pre, ei, ev, out, acc, zbuf, src_v, dst_v, val_v,
                       rows_v, sem, s,
                       set_id=set_id,
                       edge_base=s * (E // NS),
                       nchunks=E // NS // CH,
                       goff=0, goff_static=half * N,
                       slot=1 + pid)


def _spmm_final(pre, ei, ev, out, acc, zbuf, src_v, dst_v, val_v, rows_v, sem):
    c = lax.axis_index("c")
    s = lax.axis_index("s")
    _zero_fill(zbuf)
    _spmm_pass(pre, ei, ev, out, acc, zbuf, src_v, dst_v, val_v, rows_v, sem,
               s,
               set_id=5 + c,
               edge_base=s * (E // NS),
               nchunks=E // NS // CH,
               goff=c * NP, goff_static=None,
               slot=c)


def _make_spmm(body, n_slots):
    mesh = plsc.VectorSubcoreMesh(
        core_axis_name="c", subcore_axis_name="s",
        num_cores=NC, num_subcores=NS)
    return pl.kernel(
        body,
        out_type=jax.ShapeDtypeStruct((n_slots * NP, D), jnp.float32),
        mesh=mesh,
        scratch_types=[
            pltpu.VMEM_SHARED((NP, D), jnp.float32),
            pltpu.VMEM((ZROWS, D), jnp.float32),
            pltpu.VMEM((CH,), jnp.int32),
            pltpu.VMEM((CH,), jnp.int32),
            pltpu.VMEM((CH,), jnp.float32),
            pltpu.VMEM((CH, D), jnp.float32),
            pltpu.SemaphoreType.DMA,
        ],
    )


def _dense_body(x_ref, wa_ref, wb_ref, w1_ref, b1_ref, w2_ref, o_ref):
    xb = x_ref[...]
    pa = jnp.dot(xb, wa_ref[...], preferred_element_type=jnp.float32)
    pb = jnp.dot(xb, wb_ref[...], preferred_element_type=jnp.float32)
    al = pa + 0.5 * (pa * pa - pa * pb)
    w2row = w2_ref[...]
    h0 = jnp.tanh(jnp.dot(pa, w1_ref[...], preferred_element_type=jnp.float32)
                  + b1_ref[...])
    h1 = jnp.tanh(jnp.dot(al, w1_ref[...], preferred_element_type=jnp.float32)
                  + b1_ref[...])
    w0 = jnp.sum(h0 * w2row, axis=1, keepdims=True)
    w1s = jnp.sum(h1 * w2row, axis=1, keepdims=True)
    m = jnp.maximum(w0, w1s)
    e0 = jnp.exp(w0 - m)
    e1 = jnp.exp(w1s - m)
    inv = 1.0 / (e0 + e1)
    ps = (e0 * inv) * pa + (e1 * inv) * al
    o_ref[0] = ps
    o_ref[1] = ps * ps


def _bilinear_body(s1l, s1h, s2l, s2h, s3l, s3h, s4l, s4h, o_ref):
    bp1 = 0.5 * (s1l[0] * s1l[0] - s1h[0])
    bp2 = 0.5 * (s2l[0] * s2l[0] - s2h[0])
    bp3 = 0.5 * (s3l[0] * s3l[0] - s3h[0])
    bp4 = 0.5 * (s4l[0] * s4l[0] - s4h[0])
    o_ref[0] = bp1 - bp3
    o_ref[1] = bp2 - bp4


def _final_body(s0a, s0b, o5, o6, o_ref):
    gcn = s0a[0] + s0b[0]
    bi = (1.0 - BETA) * o5[0] + BETA * o6[0]
    o_ref[...] = jnp.maximum((1.0 - ALPHA) * gcn + ALPHA * bi, 0.0)


def kernel(x, edge_index, edge_vals, W_a, W_b, W_c, att_w1, att_b1, att_w2):
    del W_c
    BN = 1000
    grid = (N // BN,)

    pre_cat = pl.pallas_call(
        _dense_body,
        grid=grid,
        in_specs=[
            pl.BlockSpec((BN, D), lambda n: (n, 0)),
            pl.BlockSpec((D, D), lambda n: (0, 0)),
            pl.BlockSpec((D, D), lambda n: (0, 0)),
            pl.BlockSpec((D, ATT_H), lambda n: (0, 0)),
            pl.BlockSpec((1, ATT_H), lambda n: (0, 0)),
            pl.BlockSpec((1, ATT_H), lambda n: (0, 0)),
        ],
        out_specs=pl.BlockSpec((2, BN, D), lambda n: (0, n, 0)),
        out_shape=jax.ShapeDtypeStruct((2, N, D), jnp.float32),
    )(x, W_a, W_b, att_w1, att_b1.reshape(1, ATT_H), att_w2.reshape(1, ATT_H))

    ei_flat = edge_index.reshape(-1)
    ev_flat = edge_vals.reshape(-1)

    spmm_main = _make_spmm(_spmm_main, 10)
    s_flat = spmm_main(pre_cat.reshape(2 * N, D), ei_flat, ev_flat)
    s_all = s_flat.reshape(10, NP, D)

    BC = 1280
    cslot = lambda slot: pl.BlockSpec((1, BC, D), lambda n, s=slot: (s, n, 0))
    b_cat = pl.pallas_call(
        _bilinear_body,
        grid=(NP // BC,),
        in_specs=[cslot(k) for k in (1, 2, 3, 4, 5, 6, 7, 8)],
        out_specs=pl.BlockSpec((2, BC, D), lambda n: (0, n, 0)),
        out_shape=jax.ShapeDtypeStruct((2, NP, D), jnp.float32),
    )(*([s_all] * 8))

    spmm_final = _make_spmm(_spmm_final, 2)
    o_flat = spmm_final(b_cat.reshape(2 * NP, D), ei_flat, ev_flat)
    o_all = o_flat.reshape(2, NP, D)

    eslot = lambda slot: pl.BlockSpec((1, BN, D), lambda n, s=slot: (s, n, 0))
    out = pl.pallas_call(
        _final_body,
        grid=grid,
        in_specs=[eslot(0), eslot(9), eslot(0), eslot(1)],
        out_specs=pl.BlockSpec((BN, D), lambda n: (n, 0)),
        out_shape=jax.ShapeDtypeStruct((N, D), jnp.float32),
    )(s_all, s_all, o_all, o_all)
    return out


# speedup vs baseline: 7.0507x; 2.6369x over previous
import functools

import jax
import jax.numpy as jnp
from jax import lax
from jax.experimental import pallas as pl
from jax.experimental.pallas import tpu as pltpu
from jax.experimental.pallas import tpu_sc as plsc

N = 10000
E = 320000
D = 128
ATT_H = 32
ALPHA = 0.5
BETA = 0.5

NC = 2
NS = 16
LANES = 16

CH = 80
G = 25
NP = 10240
STRIPE = NP // NS
ZROWS = 80

def _zero_fill(zbuf):
    def body(r, _):
        for j in range(D // LANES):
            zbuf[r, pl.ds(j * LANES, LANES)] = jnp.zeros((LANES,), jnp.float32)
        return 0
    lax.fori_loop(0, ZROWS, body, 0)


def _spmm_pass(pre, ei, ev, out, acc, zbuf, src_b, dst_b, val_b, rows,
               gsems, s, set_id, edge_base, nchunks, goff, goff_static, slot):
    for z in range(STRIPE // ZROWS):
        pltpu.sync_copy(zbuf, acc.at[pl.ds(s * STRIPE + z * ZROWS, ZROWS)])

    src_off = (2 * set_id + 1) * E + edge_base
    dst_off = 2 * set_id * E + edge_base
    val_off = set_id * E + edge_base
    plsc.subcore_barrier()

    def issue_gather(c, b):
        pltpu.async_copy(pre.at[src_b.at[pl.ds(c * CH, CH)]], rows[b], gsems[b])

    def wait_gather(b):
        pltpu.make_async_copy(pre.at[src_b.at[pl.ds(0, CH)]], rows[b],
                              gsems[b]).wait()

    def super_block(sup, _):
        base = sup * (G * CH)
        pltpu.sync_copy(ei.at[pl.ds(src_off + base, G * CH)], src_b)
        pltpu.sync_copy(ei.at[pl.ds(dst_off + base, G * CH)], dst_b)
        pltpu.sync_copy(ev.at[pl.ds(val_off + base, G * CH)], val_b)

        if goff_static is None or goff_static != 0:
            add = goff if goff_static is None else goff_static

            def adj(r, _):
                sl = pl.ds(r * LANES, LANES)
                src_b[sl] = src_b[sl] + add
                return 0
            lax.fori_loop(0, G * CH // LANES, adj, 0)

        issue_gather(0, 0)
        issue_gather(1, 1)

        def group(t, _):
            for b in range(2):
                c = 2 * t + b

                @pl.when(c < G)
                def _():
                    wait_gather(b)

                    def scale(g, _):
                        vv = val_b[pl.ds(c * CH + g * LANES, LANES)]
                        for j in range(LANES):
                            v = vv[j]
                            e = g * LANES + j
                            for k in range(D // LANES):
                                sl = pl.ds(k * LANES, LANES)
                                rows[b][e, sl] = rows[b][e, sl] * v
                        return 0
                    lax.fori_loop(0, CH // LANES, scale, 0)

                    pltpu.sync_copy(rows[b],
                                    acc.at[dst_b.at[pl.ds(c * CH, CH)]],
                                    add=True)

                    @pl.when(c + 2 < G)
                    def _():
                        issue_gather(c + 2, b)
            return 0

        lax.fori_loop(0, (G + 1) // 2, group, 0)
        return 0

    lax.fori_loop(0, nchunks // G, super_block, 0)
    plsc.subcore_barrier()

    pltpu.sync_copy(acc.at[pl.ds(s * STRIPE, STRIPE)],
                    out.at[pl.ds(slot * NP + s * STRIPE, STRIPE)])


def _spmm_main(pre, ei2, ev2, out, acc, zbuf, src_b, dst_b, val_b,
               rows0, rows1, gsem0, gsem1):
    c = lax.axis_index("c")
    s = lax.axis_index("s")
    _zero_fill(zbuf)
    rows, gsems = (rows0, rows1), (gsem0, gsem1)
    for i in range(5):
        if i == 0:
            _spmm_pass(pre, ei2, ev2, out, acc, zbuf, src_b, dst_b, val_b,
                       rows, gsems, s,
                       set_id=0,
                       edge_base=c * (E // 2) + s * (E // 2 // NS),
                       nchunks=E // 2 // NS // CH,
                       goff=0, goff_static=0,
                       slot=c * 9)
        else:
            pid = c * 4 + (i - 1)
            set_id = 1 + 2 * c + (i - 1) // 2
            half = (i - 1) % 2
            _spmm_pass(pre, ei2, ev2, out, acc, zbuf, src_b, dst_b, val_b,
                       rows, gsems, s,
                       set_id=set_id,
                       edge_base=s * (E // NS),
                       nchunks=E // NS // CH,
                       goff=0, goff_static=half * N,
                       slot=1 + pid)


def _spmm_final(pre, ei2, ev2, out, acc, zbuf, src_b, dst_b, val_b,
                rows0, rows1, gsem0, gsem1):
    c = lax.axis_index("c")
    s = lax.axis_index("s")
    _zero_fill(zbuf)
    _spmm_pass(pre, ei2, ev2, out, acc, zbuf, src_b, dst_b, val_b,
               (rows0, rows1), (gsem0, gsem1), s,
               set_id=5 + c,
               edge_base=s * (E // NS),
               nchunks=E // NS // CH,
               goff=c * NP, goff_static=None,
               slot=c)


def _make_spmm(body, n_slots):
    mesh = plsc.VectorSubcoreMesh(
        core_axis_name="c", subcore_axis_name="s",
        num_cores=NC, num_subcores=NS)
    return pl.kernel(
        body,
        out_type=jax.ShapeDtypeStruct((n_slots * NP, D), jnp.float32),
        mesh=mesh,
        scratch_types=[
            pltpu.VMEM_SHARED((NP, D), jnp.float32),
            pltpu.VMEM((ZROWS, D), jnp.float32),
            pltpu.VMEM((G * CH,), jnp.int32),
            pltpu.VMEM((G * CH,), jnp.int32),
            pltpu.VMEM((G * CH,), jnp.float32),
            pltpu.VMEM((CH, D), jnp.float32),
            pltpu.VMEM((CH, D), jnp.float32),
            pltpu.SemaphoreType.DMA,
            pltpu.SemaphoreType.DMA,
        ],
    )


def _dense_body(x_ref, wa_ref, wb_ref, w1_ref, b1_ref, w2_ref, o_ref):
    xb = x_ref[...]
    pa = jnp.dot(xb, wa_ref[...], preferred_element_type=jnp.float32)
    pb = jnp.dot(xb, wb_ref[...], preferred_element_type=jnp.float32)
    al = pa + 0.5 * (pa * pa - pa * pb)
    w2row = w2_ref[...]
    h0 = jnp.tanh(jnp.dot(pa, w1_ref[...], preferred_element_type=jnp.float32)
                  + b1_ref[...])
    h1 = jnp.tanh(jnp.dot(al, w1_ref[...], preferred_element_type=jnp.float32)
                  + b1_ref[...])
    w0 = jnp.sum(h0 * w2row, axis=1, keepdims=True)
    w1s = jnp.sum(h1 * w2row, axis=1, keepdims=True)
    m = jnp.maximum(w0, w1s)
    e0 = jnp.exp(w0 - m)
    e1 = jnp.exp(w1s - m)
    inv = 1.0 / (e0 + e1)
    ps = (e0 * inv) * pa + (e1 * inv) * al
    o_ref[0] = ps
    o_ref[1] = ps * ps


def _bilinear_body(s1l, s1h, s2l, s2h, s3l, s3h, s4l, s4h, o_ref):
    bp1 = 0.5 * (s1l[0] * s1l[0] - s1h[0])
    bp2 = 0.5 * (s2l[0] * s2l[0] - s2h[0])
    bp3 = 0.5 * (s3l[0] * s3l[0] - s3h[0])
    bp4 = 0.5 * (s4l[0] * s4l[0] - s4h[0])
    o_ref[0] = bp1 - bp3
    o_ref[1] = bp2 - bp4


def _final_body(s0a, s0b, o5, o6, o_ref):
    gcn = s0a[0] + s0b[0]
    bi = (1.0 - BETA) * o5[0] + BETA * o6[0]
    o_ref[...] = jnp.maximum((1.0 - ALPHA) * gcn + ALPHA * bi, 0.0)


def kernel(x, edge_index, edge_vals, W_a, W_b, W_c, att_w1, att_b1, att_w2):
    del W_c
    BN = 1000
    grid = (N // BN,)

    pre_cat = pl.pallas_call(
        _dense_body,
        grid=grid,
        in_specs=[
            pl.BlockSpec((BN, D), lambda n: (n, 0)),
            pl.BlockSpec((D, D), lambda n: (0, 0)),
            pl.BlockSpec((D, D), lambda n: (0, 0)),
            pl.BlockSpec((D, ATT_H), lambda n: (0, 0)),
            pl.BlockSpec((1, ATT_H), lambda n: (0, 0)),
            pl.BlockSpec((1, ATT_H), lambda n: (0, 0)),
        ],
        out_specs=pl.BlockSpec((2, BN, D), lambda n: (0, n, 0)),
        out_shape=jax.ShapeDtypeStruct((2, N, D), jnp.float32),
    )(x, W_a, W_b, att_w1, att_b1.reshape(1, ATT_H), att_w2.reshape(1, ATT_H))

    ei_flat = edge_index.reshape(-1)
    ev_flat = edge_vals.reshape(-1)

    spmm_main = _make_spmm(_spmm_main, 10)
    s_flat = spmm_main(pre_cat.reshape(2 * N, D), ei_flat, ev_flat)
    s_all = s_flat.reshape(10, NP, D)

    BC = 1280
    cslot = lambda slot: pl.BlockSpec((1, BC, D), lambda n, s=slot: (s, n, 0))
    b_cat = pl.pallas_call(
        _bilinear_body,
        grid=(NP // BC,),
        in_specs=[cslot(k) for k in (1, 2, 3, 4, 5, 6, 7, 8)],
        out_specs=pl.BlockSpec((2, BC, D), lambda n: (0, n, 0)),
        out_shape=jax.ShapeDtypeStruct((2, NP, D), jnp.float32),
    )(*([s_all] * 8))

    spmm_final = _make_spmm(_spmm_final, 2)
    o_flat = spmm_final(b_cat.reshape(2 * NP, D), ei_flat, ev_flat)
    o_all = o_flat.reshape(2, NP, D)

    eslot = lambda slot: pl.BlockSpec((1, BN, D), lambda n, s=slot: (s, n, 0))
    out = pl.pallas_call(
        _final_body,
        grid=grid,
        in_specs=[eslot(0), eslot(9), eslot(0), eslot(1)],
        out_specs=pl.BlockSpec((BN, D), lambda n: (n, 0)),
        out_shape=jax.ShapeDtypeStruct((N, D), jnp.float32),
    )(s_all, s_all, o_all, o_all)
    return out


# speedup vs baseline: 8.2644x; 1.1721x over previous
import functools

import jax
import jax.numpy as jnp
from jax import lax
from jax.experimental import pallas as pl
from jax.experimental.pallas import tpu as pltpu
from jax.experimental.pallas import tpu_sc as plsc

N = 10000
E = 320000
D = 128
ATT_H = 32
ALPHA = 0.5
BETA = 0.5

NC = 2
NS = 16
LANES = 16

CH = 80
G = 25
NB = 4
NP = 10240
STRIPE = NP // NS

def _spmm_pass(pre, ei, ev, zrows, out, acc, src_b, dst_b, val_b, rows,
               gsems, ssems, msem, s, set_id, edge_base, nsup, goff,
               goff_static, slot):
    pltpu.sync_copy(zrows, acc.at[pl.ds(s * STRIPE, STRIPE)])

    src_off = (2 * set_id + 1) * E + edge_base
    dst_off = 2 * set_id * E + edge_base
    val_off = set_id * E + edge_base
    plsc.subcore_barrier()

    def issue_gather(c, b):
        pltpu.async_copy(pre.at[src_b.at[pl.ds(c * CH, CH)]], rows[b], gsems[b])

    def wait_gather(b):
        pltpu.make_async_copy(pre.at[src_b.at[pl.ds(0, CH)]], rows[b],
                              gsems[b]).wait()

    def issue_scatter(c, b):
        pltpu.async_copy(rows[b], acc.at[dst_b.at[pl.ds(c * CH, CH)]],
                         ssems[b], add=True)

    def wait_scatter(b):
        pltpu.make_async_copy(rows[b], acc.at[dst_b.at[pl.ds(0, CH)]],
                              ssems[b]).wait()

    def scale_chunk(c, b):
        def scale(g, _):
            vv = val_b[pl.ds(c * CH + g * LANES, LANES)]
            for j in range(LANES):
                v = vv[j]
                e = g * LANES + j
                for k in range(D // LANES):
                    sl = pl.ds(k * LANES, LANES)
                    rows[b][e, sl] = rows[b][e, sl] * v
            return 0
        lax.fori_loop(0, CH // LANES, scale, 0)

    def super_block(sup, _):
        base = sup * (G * CH)
        d1 = pltpu.async_copy(ei.at[pl.ds(src_off + base, G * CH)], src_b, msem)
        d2 = pltpu.async_copy(ei.at[pl.ds(dst_off + base, G * CH)], dst_b, msem)
        d3 = pltpu.async_copy(ev.at[pl.ds(val_off + base, G * CH)], val_b, msem)
        d1.wait(); d2.wait(); d3.wait()

        if goff_static is None or goff_static != 0:
            add = goff if goff_static is None else goff_static

            def adj(r, _):
                sl = pl.ds(r * LANES, LANES)
                src_b[sl] = src_b[sl] + add
                return 0
            lax.fori_loop(0, G * CH // LANES, adj, 0)

        issue_gather(0, 0)
        issue_gather(1, 1)

        def group(t, _):
            for b in range(NB):
                c = NB * t + b
                wait_gather(b)
                scale_chunk(c, b)
                issue_scatter(c, b)
                nb = (b + 2) % NB
                if b < 2:
                    @pl.when(t > 0)
                    def _():
                        wait_scatter(nb)
                    issue_gather(c + 2, nb)
                elif b == 2:
                    wait_scatter(nb)
                    issue_gather(c + 2, nb)
                else:
                    @pl.when(t < G // NB - 1)
                    def _():
                        wait_scatter(nb)
                        issue_gather(c + 2, nb)
            return 0

        lax.fori_loop(0, G // NB, group, 0)

        wait_gather(0)
        scale_chunk(G - 1, 0)
        issue_scatter(G - 1, 0)
        for b in (1, 2, 3, 0):
            wait_scatter(b)
        return 0

    lax.fori_loop(0, nsup, super_block, 0)
    plsc.subcore_barrier()

    pltpu.sync_copy(acc.at[pl.ds(s * STRIPE, STRIPE)],
                    out.at[pl.ds(slot * NP + s * STRIPE, STRIPE)])


def _spmm_main(pre, ei, ev, zrows, out, acc, src_b, dst_b, val_b,
               r0, r1, r2, r3, g0, g1, g2, g3, s0, s1, s2, s3, msem):
    c = lax.axis_index("c")
    s = lax.axis_index("s")
    rows, gsems, ssems = (r0, r1, r2, r3), (g0, g1, g2, g3), (s0, s1, s2, s3)
    for i in range(5):
        if i == 0:
            _spmm_pass(pre, ei, ev, zrows, out, acc, src_b, dst_b, val_b,
                       rows, gsems, ssems, msem, s,
                       set_id=0,
                       edge_base=c * (E // 2) + s * (E // 2 // NS),
                       nsup=E // 2 // NS // CH // G,
                       goff=0, goff_static=0,
                       slot=c * 9)
        else:
            pid = c * 4 + (i - 1)
            set_id = 1 + 2 * c + (i - 1) // 2
            half = (i - 1) % 2
            _spmm_pass(pre, ei, ev, zrows, out, acc, src_b, dst_b, val_b,
                       rows, gsems, ssems, msem, s,
                       set_id=set_id,
                       edge_base=s * (E // NS),
                       nsup=E // NS // CH // G,
                       goff=0, goff_static=half * N,
                       slot=1 + pid)


def _spmm_final(pre, ei, ev, zrows, out, acc, src_b, dst_b, val_b,
                r0, r1, r2, r3, g0, g1, g2, g3, s0, s1, s2, s3, msem):
    c = lax.axis_index("c")
    s = lax.axis_index("s")
    _spmm_pass(pre, ei, ev, zrows, out, acc, src_b, dst_b, val_b,
               (r0, r1, r2, r3), (g0, g1, g2, g3), (s0, s1, s2, s3), msem, s,
               set_id=5 + c,
               edge_base=s * (E // NS),
               nsup=E // NS // CH // G,
               goff=c * NP, goff_static=None,
               slot=c)


def _make_spmm(body, n_slots):
    mesh = plsc.VectorSubcoreMesh(
        core_axis_name="c", subcore_axis_name="s",
        num_cores=NC, num_subcores=NS)
    return pl.kernel(
        body,
        out_type=jax.ShapeDtypeStruct((n_slots * NP, D), jnp.float32),
        mesh=mesh,
        scratch_types=[
            pltpu.VMEM_SHARED((NP, D), jnp.float32),
            pltpu.VMEM((G * CH,), jnp.int32),
            pltpu.VMEM((G * CH,), jnp.int32),
            pltpu.VMEM((G * CH,), jnp.float32),
        ] + [pltpu.VMEM((CH, D), jnp.float32) for _ in range(NB)]
          + [pltpu.SemaphoreType.DMA for _ in range(2 * NB + 1)],
    )


def _dense_body(x_ref, wa_ref, wb_ref, w1_ref, b1_ref, w2_ref, o_ref):
    xb = x_ref[...]
    pa = jnp.dot(xb, wa_ref[...], preferred_element_type=jnp.float32)
    pb = jnp.dot(xb, wb_ref[...], preferred_element_type=jnp.float32)
    al = pa + 0.5 * (pa * pa - pa * pb)
    w2row = w2_ref[...]
    h0 = jnp.tanh(jnp.dot(pa, w1_ref[...], preferred_element_type=jnp.float32)
                  + b1_ref[...])
    h1 = jnp.tanh(jnp.dot(al, w1_ref[...], preferred_element_type=jnp.float32)
                  + b1_ref[...])
    w0 = jnp.sum(h0 * w2row, axis=1, keepdims=True)
    w1s = jnp.sum(h1 * w2row, axis=1, keepdims=True)
    m = jnp.maximum(w0, w1s)
    e0 = jnp.exp(w0 - m)
    e1 = jnp.exp(w1s - m)
    inv = 1.0 / (e0 + e1)
    ps = (e0 * inv) * pa + (e1 * inv) * al
    o_ref[0] = ps
    o_ref[1] = ps * ps


def _bilinear_body(s1l, s1h, s2l, s2h, s3l, s3h, s4l, s4h, o_ref):
    bp1 = 0.5 * (s1l[0] * s1l[0] - s1h[0])
    bp2 = 0.5 * (s2l[0] * s2l[0] - s2h[0])
    bp3 = 0.5 * (s3l[0] * s3l[0] - s3h[0])
    bp4 = 0.5 * (s4l[0] * s4l[0] - s4h[0])
    o_ref[0] = bp1 - bp3
    o_ref[1] = bp2 - bp4


def _final_body(s0a, s0b, o5, o6, o_ref):
    gcn = s0a[0] + s0b[0]
    bi = (1.0 - BETA) * o5[0] + BETA * o6[0]
    o_ref[...] = jnp.maximum((1.0 - ALPHA) * gcn + ALPHA * bi, 0.0)


def kernel(x, edge_index, edge_vals, W_a, W_b, W_c, att_w1, att_b1, att_w2):
    del W_c
    BN = 1000
    grid = (N // BN,)

    pre_cat = pl.pallas_call(
        _dense_body,
        grid=grid,
        in_specs=[
            pl.BlockSpec((BN, D), lambda n: (n, 0)),
            pl.BlockSpec((D, D), lambda n: (0, 0)),
            pl.BlockSpec((D, D), lambda n: (0, 0)),
            pl.BlockSpec((D, ATT_H), lambda n: (0, 0)),
            pl.BlockSpec((1, ATT_H), lambda n: (0, 0)),
            pl.BlockSpec((1, ATT_H), lambda n: (0, 0)),
        ],
        out_specs=pl.BlockSpec((2, BN, D), lambda n: (0, n, 0)),
        out_shape=jax.ShapeDtypeStruct((2, N, D), jnp.float32),
    )(x, W_a, W_b, att_w1, att_b1.reshape(1, ATT_H), att_w2.reshape(1, ATT_H))

    ei_flat = edge_index.reshape(-1)
    ev_flat = edge_vals.reshape(-1)

    zrows = jnp.zeros((STRIPE, D), jnp.float32)

    spmm_main = _make_spmm(_spmm_main, 10)
    s_flat = spmm_main(pre_cat.reshape(2 * N, D), ei_flat, ev_flat, zrows)
    s_all = s_flat.reshape(10, NP, D)

    BC = 1280
    cslot = lambda slot: pl.BlockSpec((1, BC, D), lambda n, s=slot: (s, n, 0))
    b_cat = pl.pallas_call(
        _bilinear_body,
        grid=(NP // BC,),
        in_specs=[cslot(k) for k in (1, 2, 3, 4, 5, 6, 7, 8)],
        out_specs=pl.BlockSpec((2, BC, D), lambda n: (0, n, 0)),
        out_shape=jax.ShapeDtypeStruct((2, NP, D), jnp.float32),
    )(*([s_all] * 8))

    spmm_final = _make_spmm(_spmm_final, 2)
    o_flat = spmm_final(b_cat.reshape(2 * NP, D), ei_flat, ev_flat, zrows)
    o_all = o_flat.reshape(2, NP, D)

    eslot = lambda slot: pl.BlockSpec((1, BN, D), lambda n, s=slot: (s, n, 0))
    out = pl.pallas_call(
        _final_body,
        grid=grid,
        in_specs=[eslot(0), eslot(9), eslot(0), eslot(1)],
        out_specs=pl.BlockSpec((BN, D), lambda n: (n, 0)),
        out_shape=jax.ShapeDtypeStruct((N, D), jnp.float32),
    )(s_all, s_all, o_all, o_all)
    return out


# speedup vs baseline: 9.0015x; 1.0892x over previous
import functools

import jax
import jax.numpy as jnp
from jax import lax
from jax.experimental import pallas as pl
from jax.experimental.pallas import tpu as pltpu
from jax.experimental.pallas import tpu_sc as plsc

N = 10000
E = 320000
D = 128
ATT_H = 32
ALPHA = 0.5
BETA = 0.5

NC = 2
NS = 16
LANES = 16

CH = 40
G = 50
NB = 8
LEAD = 6
NP = 10240
STRIPE = NP // NS

def _spmm_pass(pre, ei, ev, zrows, out, acc, src_b, dst_b, val_b, rows,
               gsems, ssems, msem, s, set_id, edge_base, nsup, goff,
               goff_static, slot):
    pltpu.sync_copy(zrows, acc.at[pl.ds(s * STRIPE, STRIPE)])

    src_off = (2 * set_id + 1) * E + edge_base
    dst_off = 2 * set_id * E + edge_base
    val_off = set_id * E + edge_base
    plsc.subcore_barrier()

    def issue_gather(c, b):
        pltpu.async_copy(pre.at[src_b.at[pl.ds(c * CH, CH)]], rows[b], gsems[b])

    def wait_gather(b):
        pltpu.make_async_copy(pre.at[src_b.at[pl.ds(0, CH)]], rows[b],
                              gsems[b]).wait()

    def issue_scatter(c, b):
        pltpu.async_copy(rows[b], acc.at[dst_b.at[pl.ds(c * CH, CH)]],
                         ssems[b], add=True)

    def wait_scatter(b):
        pltpu.make_async_copy(rows[b], acc.at[dst_b.at[pl.ds(0, CH)]],
                              ssems[b]).wait()

    def scale_chunk(c, b):
        def scale(g, _):
            vv = val_b[pl.ds(c * CH + g * LANES, LANES)]
            for j in range(LANES):
                v = vv[j]
                e = g * LANES + j
                for k in range(D // LANES):
                    sl = pl.ds(k * LANES, LANES)
                    rows[b][e, sl] = rows[b][e, sl] * v
            return 0
        lax.fori_loop(0, CH // LANES, scale, 0)

    def super_block(sup, _):
        base = sup * (G * CH)
        d1 = pltpu.async_copy(ei.at[pl.ds(src_off + base, G * CH)], src_b, msem)
        d2 = pltpu.async_copy(ei.at[pl.ds(dst_off + base, G * CH)], dst_b, msem)
        d3 = pltpu.async_copy(ev.at[pl.ds(val_off + base, G * CH)], val_b, msem)
        d1.wait(); d2.wait(); d3.wait()

        if goff_static is None or goff_static != 0:
            add = goff if goff_static is None else goff_static

            def adj(r, _):
                sl = pl.ds(r * LANES, LANES)
                src_b[sl] = src_b[sl] + add
                return 0
            lax.fori_loop(0, G * CH // LANES, adj, 0)

        for b in range(LEAD):
            issue_gather(b, b)

        def group(t, _):
            for b in range(NB):
                c = NB * t + b
                wait_gather(b)
                nb = (b + LEAD) % NB
                if b < 2:
                    @pl.when(t > 0)
                    def _():
                        wait_scatter(nb)
                        issue_gather(c + LEAD, nb)
                    @pl.when(t == 0)
                    def _():
                        issue_gather(c + LEAD, nb)
                elif b < 4:
                    wait_scatter(nb)
                    issue_gather(c + LEAD, nb)
                else:
                    @pl.when(t < G // NB - 1)
                    def _():
                        wait_scatter(nb)
                        issue_gather(c + LEAD, nb)
                scale_chunk(c, b)
                issue_scatter(c, b)
            return 0

        lax.fori_loop(0, G // NB, group, 0)

        for j in range(2):
            wait_gather(j)
            scale_chunk(G - 2 + j, j)
            issue_scatter(G - 2 + j, j)
        for b in (2, 3, 4, 5, 6, 7, 0, 1):
            wait_scatter(b)
        return 0

    lax.fori_loop(0, nsup, super_block, 0)
    plsc.subcore_barrier()

    pltpu.sync_copy(acc.at[pl.ds(s * STRIPE, STRIPE)],
                    out.at[pl.ds(slot * NP + s * STRIPE, STRIPE)])


def _spmm_main(pre, ei, ev, zrows, out, acc, src_b, dst_b, val_b, *bufs):
    c = lax.axis_index("c")
    s = lax.axis_index("s")
    rows, gsems, ssems, msem = (bufs[:NB], bufs[NB:2 * NB],
                                bufs[2 * NB:3 * NB], bufs[3 * NB])
    for i in range(5):
        if i == 0:
            _spmm_pass(pre, ei, ev, zrows, out, acc, src_b, dst_b, val_b,
                       rows, gsems, ssems, msem, s,
                       set_id=0,
                       edge_base=c * (E // 2) + s * (E // 2 // NS),
                       nsup=E // 2 // NS // CH // G,
                       goff=0, goff_static=0,
                       slot=c * 9)
        else:
            pid = c * 4 + (i - 1)
            set_id = 1 + 2 * c + (i - 1) // 2
            half = (i - 1) % 2
            _spmm_pass(pre, ei, ev, zrows, out, acc, src_b, dst_b, val_b,
                       rows, gsems, ssems, msem, s,
                       set_id=set_id,
                       edge_base=s * (E // NS),
                       nsup=E // NS // CH // G,
                       goff=0, goff_static=half * N,
                       slot=1 + pid)


def _spmm_final(pre, ei, ev, zrows, out, acc, src_b, dst_b, val_b, *bufs):
    c = lax.axis_index("c")
    s = lax.axis_index("s")
    rows, gsems, ssems, msem = (bufs[:NB], bufs[NB:2 * NB],
                                bufs[2 * NB:3 * NB], bufs[3 * NB])
    _spmm_pass(pre, ei, ev, zrows, out, acc, src_b, dst_b, val_b,
               rows, gsems, ssems, msem, s,
               set_id=5 + c,
               edge_base=s * (E // NS),
               nsup=E // NS // CH // G,
               goff=c * NP, goff_static=None,
               slot=c)


def _make_spmm(body, n_slots):
    mesh = plsc.VectorSubcoreMesh(
        core_axis_name="c", subcore_axis_name="s",
        num_cores=NC, num_subcores=NS)
    return pl.kernel(
        body,
        out_type=jax.ShapeDtypeStruct((n_slots * NP, D), jnp.float32),
        mesh=mesh,
        scratch_types=[
            pltpu.VMEM_SHARED((NP, D), jnp.float32),
            pltpu.VMEM((G * CH,), jnp.int32),
            pltpu.VMEM((G * CH,), jnp.int32),
            pltpu.VMEM((G * CH,), jnp.float32),
        ] + [pltpu.VMEM((CH, D), jnp.float32) for _ in range(NB)]
          + [pltpu.SemaphoreType.DMA for _ in range(2 * NB + 1)],
    )


def _dense_body(x_ref, wa_ref, wb_ref, w1_ref, b1_ref, w2_ref, o_ref):
    xb = x_ref[...]
    pa = jnp.dot(xb, wa_ref[...], preferred_element_type=jnp.float32)
    pb = jnp.dot(xb, wb_ref[...], preferred_element_type=jnp.float32)
    al = pa + 0.5 * (pa * pa - pa * pb)
    w2row = w2_ref[...]
    h0 = jnp.tanh(jnp.dot(pa, w1_ref[...], preferred_element_type=jnp.float32)
                  + b1_ref[...])
    h1 = jnp.tanh(jnp.dot(al, w1_ref[...], preferred_element_type=jnp.float32)
                  + b1_ref[...])
    w0 = jnp.sum(h0 * w2row, axis=1, keepdims=True)
    w1s = jnp.sum(h1 * w2row, axis=1, keepdims=True)
    m = jnp.maximum(w0, w1s)
    e0 = jnp.exp(w0 - m)
    e1 = jnp.exp(w1s - m)
    inv = 1.0 / (e0 + e1)
    ps = (e0 * inv) * pa + (e1 * inv) * al
    o_ref[0] = ps
    o_ref[1] = ps * ps


def _bilinear_body(s1l, s1h, s2l, s2h, s3l, s3h, s4l, s4h, o_ref):
    bp1 = 0.5 * (s1l[0] * s1l[0] - s1h[0])
    bp2 = 0.5 * (s2l[0] * s2l[0] - s2h[0])
    bp3 = 0.5 * (s3l[0] * s3l[0] - s3h[0])
    bp4 = 0.5 * (s4l[0] * s4l[0] - s4h[0])
    o_ref[0] = bp1 - bp3
    o_ref[1] = bp2 - bp4


def _final_body(s0a, s0b, o5, o6, o_ref):
    gcn = s0a[0] + s0b[0]
    bi = (1.0 - BETA) * o5[0] + BETA * o6[0]
    o_ref[...] = jnp.maximum((1.0 - ALPHA) * gcn + ALPHA * bi, 0.0)


def kernel(x, edge_index, edge_vals, W_a, W_b, W_c, att_w1, att_b1, att_w2):
    del W_c
    BN = 1000
    grid = (N // BN,)

    pre_cat = pl.pallas_call(
        _dense_body,
        grid=grid,
        in_specs=[
            pl.BlockSpec((BN, D), lambda n: (n, 0)),
            pl.BlockSpec((D, D), lambda n: (0, 0)),
            pl.BlockSpec((D, D), lambda n: (0, 0)),
            pl.BlockSpec((D, ATT_H), lambda n: (0, 0)),
            pl.BlockSpec((1, ATT_H), lambda n: (0, 0)),
            pl.BlockSpec((1, ATT_H), lambda n: (0, 0)),
        ],
        out_specs=pl.BlockSpec((2, BN, D), lambda n: (0, n, 0)),
        out_shape=jax.ShapeDtypeStruct((2, N, D), jnp.float32),
    )(x, W_a, W_b, att_w1, att_b1.reshape(1, ATT_H), att_w2.reshape(1, ATT_H))

    ei_flat = edge_index.reshape(-1)
    ev_flat = edge_vals.reshape(-1)

    zrows = jnp.zeros((STRIPE, D), jnp.float32)

    spmm_main = _make_spmm(_spmm_main, 10)
    s_flat = spmm_main(pre_cat.reshape(2 * N, D), ei_flat, ev_flat, zrows)
    s_all = s_flat.reshape(10, NP, D)

    BC = 1280
    cslot = lambda slot: pl.BlockSpec((1, BC, D), lambda n, s=slot: (s, n, 0))
    b_cat = pl.pallas_call(
        _bilinear_body,
        grid=(NP // BC,),
        in_specs=[cslot(k) for k in (1, 2, 3, 4, 5, 6, 7, 8)],
        out_specs=pl.BlockSpec((2, BC, D), lambda n: (0, n, 0)),
        out_shape=jax.ShapeDtypeStruct((2, NP, D), jnp.float32),
    )(*([s_all] * 8))

    spmm_final = _make_spmm(_spmm_final, 2)
    o_flat = spmm_final(b_cat.reshape(2 * NP, D), ei_flat, ev_flat, zrows)
    o_all = o_flat.reshape(2, NP, D)

    eslot = lambda slot: pl.BlockSpec((1, BN, D), lambda n, s=slot: (s, n, 0))
    out = pl.pallas_call(
        _final_body,
        grid=grid,
        in_specs=[eslot(0), eslot(9), eslot(0), eslot(1)],
        out_specs=pl.BlockSpec((BN, D), lambda n: (n, 0)),
        out_shape=jax.ShapeDtypeStruct((N, D), jnp.float32),
    )(s_all, s_all, o_all, o_all)
    return out
